# Initial kernel scaffold; baseline (speedup 1.0000x reference)
import jax
import jax.numpy as jnp
from jax.experimental import pallas as pl


def kernel(x_user, x_item, edge_index_u2i, edge_index_i2u, edge_label_index, W_emb_user, b_emb_user, W_emb_item, b_emb_item, W1, b1, W2, b2):
    raise NotImplementedError("write your pallas kernel here")


# speedup vs baseline: 70.1460x; 70.1460x over previous
import functools

import jax
import jax.numpy as jnp
from jax import lax
from jax.experimental import pallas as pl
from jax.experimental.pallas import tpu as pltpu
from jax.experimental.pallas import tpu_sc as plsc

NU = 10000
NI = 10000
E = 320000
EQ = 100000
D = 128
H = 64

NC = 2
NS = 16
NW = NC * NS
L = 16
CHUNK = E // NW

EQ_ROWS = 782
EQ_PAD = EQ_ROWS * 128


def _hist_body(eu_hbm, ei_hbm, out_u, out_i, idx_v, cu_v, ci_v):
    wid = lax.axis_index("c") * NS + lax.axis_index("s")
    zeros16 = jnp.zeros((L,), jnp.float32)
    ones16 = jnp.ones((L,), jnp.float32)

    def zero_body(i, carry):
        cu_v[pl.ds(i * L, L)] = zeros16
        ci_v[pl.ds(i * L, L)] = zeros16
        return carry

    lax.fori_loop(0, NU // L, zero_body, 0)

    base = wid * CHUNK
    pltpu.sync_copy(eu_hbm.at[pl.ds(base, CHUNK)], idx_v)

    def add_u(i, carry):
        iv = idx_v[pl.ds(i * L, L)]
        plsc.addupdate_scatter(cu_v, [iv], ones16)
        return carry

    lax.fori_loop(0, CHUNK // L, add_u, 0)

    pltpu.sync_copy(ei_hbm.at[pl.ds(base, CHUNK)], idx_v)

    def add_i(i, carry):
        iv = idx_v[pl.ds(i * L, L)]
        plsc.addupdate_scatter(ci_v, [iv], ones16)
        return carry

    lax.fori_loop(0, CHUNK // L, add_i, 0)

    pltpu.sync_copy(cu_v, out_u.at[wid])
    pltpu.sync_copy(ci_v, out_i.at[wid])


@functools.cache
def _hist():
    return pl.kernel(
        _hist_body,
        mesh=plsc.VectorSubcoreMesh(core_axis_name="c", subcore_axis_name="s"),
        out_type=[
            jax.ShapeDtypeStruct((NW, NU), jnp.float32),
            jax.ShapeDtypeStruct((NW, NI), jnp.float32),
        ],
        scratch_types=[
            pltpu.VMEM((CHUNK,), jnp.int32),
            pltpu.VMEM((NU,), jnp.float32),
            pltpu.VMEM((NI,), jnp.float32),
        ],
        compiler_params=pltpu.CompilerParams(
            use_tc_tiling_on_sc=False,
            needs_layout_passes=False,
        ),
    )


def _dense_body(pu_ref, pi_ref, xu_ref, xi_ref, weu_ref, beu_ref, wei_ref,
                bei_ref, w1_ref, b1_ref, w2_ref, b2_ref, src_ref, dst_ref,
                out_ref):
    inv_e = jnp.float32(1.0 / E)
    cu = jnp.sum(pu_ref[...], axis=0)
    ci = jnp.sum(pi_ref[...], axis=0)

    mean_user = jnp.sum(xu_ref[...] * cu[:, None], axis=0, keepdims=True) * inv_e
    mean_item = jnp.sum(xi_ref[...] * ci[:, None], axis=0, keepdims=True) * inv_e

    dot = functools.partial(
        lax.dot_general,
        dimension_numbers=(((1,), (0,)), ((), ())),
        preferred_element_type=jnp.float32,
        precision=lax.Precision.HIGHEST,
    )
    msg_i1 = dot(mean_user, weu_ref[...]) + beu_ref[...]
    msg_u1 = dot(mean_item, wei_ref[...]) + bei_ref[...]

    p_u = lax.slice(ci, (0,), (1,)).reshape(1, 1) * inv_e
    p_i = lax.slice(cu, (0,), (1,)).reshape(1, 1) * inv_e
    scale = p_u * p_i
    u_vec = scale * jnp.maximum(msg_u1, 0.0)
    i_vec = scale * jnp.maximum(msg_i1, 0.0)

    z = jnp.zeros((1, H), jnp.float32)
    combos = jnp.concatenate(
        [
            jnp.concatenate([z, z], axis=1),
            jnp.concatenate([z, i_vec], axis=1),
            jnp.concatenate([u_vec, z], axis=1),
            jnp.concatenate([u_vec, i_vec], axis=1),
        ],
        axis=0,
    )
    hid = jnp.maximum(dot(combos, w1_ref[...]) + b1_ref[...], 0.0)
    vals = jax.nn.sigmoid(dot(hid, w2_ref[...]) + b2_ref[...])

    v00 = lax.slice(vals, (0, 0), (1, 1))
    v01 = lax.slice(vals, (1, 0), (2, 1))
    v10 = lax.slice(vals, (2, 0), (3, 1))
    v11 = lax.slice(vals, (3, 0), (4, 1))

    s_mask = src_ref[...] == 0
    d_mask = dst_ref[...] == 0
    out_ref[...] = jnp.where(
        s_mask,
        jnp.where(d_mask, v11, v10),
        jnp.where(d_mask, v01, v00),
    )


_dense = pl.pallas_call(
    _dense_body,
    out_shape=jax.ShapeDtypeStruct((EQ_ROWS, 128), jnp.float32),
)


def kernel(x_user, x_item, edge_index_u2i, edge_index_i2u, edge_label_index,
           W_emb_user, b_emb_user, W_emb_item, b_emb_item, W1, b1, W2, b2):
    eu_src = edge_index_u2i[0].astype(jnp.int32)
    ei_src = edge_index_i2u[0].astype(jnp.int32)

    part_u, part_i = _hist()(eu_src, ei_src)

    eli = edge_label_index.astype(jnp.int32)
    pad = jnp.ones((2, EQ_PAD - EQ), jnp.int32)
    eli_p = jnp.concatenate([eli, pad], axis=1).reshape(2, EQ_ROWS, 128)

    out2d = _dense(
        part_u, part_i,
        x_user, x_item,
        W_emb_user, b_emb_user.reshape(1, H),
        W_emb_item, b_emb_item.reshape(1, H),
        W1, b1.reshape(1, H),
        W2, b2.reshape(1, 1),
        eli_p[0], eli_p[1],
    )
    return out2d.reshape(EQ_PAD)[:EQ]


# speedup vs baseline: 97.8412x; 1.3948x over previous
import functools

import jax
import jax.numpy as jnp
from jax import lax
from jax.experimental import pallas as pl
from jax.experimental.pallas import tpu as pltpu
from jax.experimental.pallas import tpu_sc as plsc

NU = 10000
NI = 10000
E = 320000
EQ = 100000
D = 128
H = 64

NC = 2
NS = 16
NW = NC * NS
L = 16
CHUNK = E // NW

EQ_ROWS = 782
EQ_PAD = EQ_ROWS * 128


def _hist_body(eu_hbm, ei_hbm, out_u, out_i, idx_v, cu_v, ci_v):
    wid = lax.axis_index("c") * NS + lax.axis_index("s")
    zeros16 = jnp.zeros((L,), jnp.float32)
    ones16 = jnp.ones((L,), jnp.float32)

    def zero_body(i, carry):
        cu_v[pl.ds(i * L, L)] = zeros16
        ci_v[pl.ds(i * L, L)] = zeros16
        return carry

    lax.fori_loop(0, NU // L, zero_body, 0)

    base = wid * CHUNK
    pltpu.sync_copy(eu_hbm.at[0, pl.ds(base, CHUNK)], idx_v)

    def add_u(i, carry):
        iv = idx_v[pl.ds(i * L, L)]
        plsc.addupdate_scatter(cu_v, [iv], ones16)
        return carry

    lax.fori_loop(0, CHUNK // L, add_u, 0)

    pltpu.sync_copy(ei_hbm.at[0, pl.ds(base, CHUNK)], idx_v)

    def add_i(i, carry):
        iv = idx_v[pl.ds(i * L, L)]
        plsc.addupdate_scatter(ci_v, [iv], ones16)
        return carry

    lax.fori_loop(0, CHUNK // L, add_i, 0)

    pltpu.sync_copy(cu_v, out_u.at[wid])
    pltpu.sync_copy(ci_v, out_i.at[wid])


@functools.cache
def _hist():
    return pl.kernel(
        _hist_body,
        mesh=plsc.VectorSubcoreMesh(core_axis_name="c", subcore_axis_name="s"),
        out_type=[
            jax.ShapeDtypeStruct((NW, NU), jnp.float32),
            jax.ShapeDtypeStruct((NW, NI), jnp.float32),
        ],
        scratch_types=[
            pltpu.VMEM((CHUNK,), jnp.int32),
            pltpu.VMEM((NU,), jnp.float32),
            pltpu.VMEM((NI,), jnp.float32),
        ],
        compiler_params=pltpu.CompilerParams(
            use_tc_tiling_on_sc=False,
            needs_layout_passes=False,
        ),
    )


def _dense_body(pu_ref, pi_ref, xu_ref, xi_ref, weu_ref, beu_ref, wei_ref,
                bei_ref, w1_ref, b1_ref, w2_ref, b2_ref, src_ref, dst_ref,
                out_ref):
    inv_e = jnp.float32(1.0 / E)
    cu = jnp.sum(pu_ref[...], axis=0)
    ci = jnp.sum(pi_ref[...], axis=0)

    mean_user = jnp.sum(xu_ref[...] * cu[:, None], axis=0, keepdims=True) * inv_e
    mean_item = jnp.sum(xi_ref[...] * ci[:, None], axis=0, keepdims=True) * inv_e

    dot = functools.partial(
        lax.dot_general,
        dimension_numbers=(((1,), (0,)), ((), ())),
        preferred_element_type=jnp.float32,
        precision=lax.Precision.HIGHEST,
    )
    msg_i1 = dot(mean_user, weu_ref[...]) + beu_ref[...]
    msg_u1 = dot(mean_item, wei_ref[...]) + bei_ref[...]

    p_u = lax.slice(ci, (0,), (1,)).reshape(1, 1) * inv_e
    p_i = lax.slice(cu, (0,), (1,)).reshape(1, 1) * inv_e
    scale = p_u * p_i
    u_vec = scale * jnp.maximum(msg_u1, 0.0)
    i_vec = scale * jnp.maximum(msg_i1, 0.0)

    z = jnp.zeros((1, H), jnp.float32)
    combos = jnp.concatenate(
        [
            jnp.concatenate([z, z], axis=1),
            jnp.concatenate([z, i_vec], axis=1),
            jnp.concatenate([u_vec, z], axis=1),
            jnp.concatenate([u_vec, i_vec], axis=1),
        ],
        axis=0,
    )
    hid = jnp.maximum(dot(combos, w1_ref[...]) + b1_ref[...], 0.0)
    vals = jax.nn.sigmoid(dot(hid, w2_ref[...]) + b2_ref[...])

    v00 = lax.slice(vals, (0, 0), (1, 1))
    v01 = lax.slice(vals, (1, 0), (2, 1))
    v10 = lax.slice(vals, (2, 0), (3, 1))
    v11 = lax.slice(vals, (3, 0), (4, 1))

    s_mask = src_ref[...] == 0
    d_mask = dst_ref[...] == 0
    out_ref[...] = jnp.where(
        s_mask,
        jnp.where(d_mask, v11, v10),
        jnp.where(d_mask, v01, v00),
    )


_dense = pl.pallas_call(
    _dense_body,
    out_shape=jax.ShapeDtypeStruct((EQ_ROWS, 128), jnp.float32),
)


def kernel(x_user, x_item, edge_index_u2i, edge_index_i2u, edge_label_index,
           W_emb_user, b_emb_user, W_emb_item, b_emb_item, W1, b1, W2, b2):
    part_u, part_i = _hist()(edge_index_u2i.astype(jnp.int32),
                             edge_index_i2u.astype(jnp.int32))

    eli = edge_label_index.astype(jnp.int32)
    pad = jnp.ones((2, EQ_PAD - EQ), jnp.int32)
    eli_p = jnp.concatenate([eli, pad], axis=1).reshape(2, EQ_ROWS, 128)

    out2d = _dense(
        part_u, part_i,
        x_user, x_item,
        W_emb_user, b_emb_user.reshape(1, H),
        W_emb_item, b_emb_item.reshape(1, H),
        W1, b1.reshape(1, H),
        W2, b2.reshape(1, 1),
        eli_p[0], eli_p[1],
    )
    return out2d.reshape(EQ_PAD)[:EQ]
